# baseline (speedup 1.0000x reference)
import functools

import jax
import jax.numpy as jnp
from jax import lax
from jax.experimental import pallas as pl
from jax.experimental.pallas import tpu as pltpu
from jax.experimental.pallas import tpu_sc as plsc

N = 10000
E = 320000
D = 128
G = 64

NP = 10240
SL = NP // 16
NW = 32
C = 64
K = 160
GK = 16
CD = 128
KD = K * C // CD
EPAD = NW * K * C
BLK = 1024
NB = NP // BLK

_mesh = plsc.VectorSubcoreMesh(core_axis_name="c", subcore_axis_name="s")
_sc_params = pltpu.CompilerParams(needs_layout_passes=False)


@functools.partial(
    pl.kernel,
    out_type=jax.ShapeDtypeStruct((2, NP), jnp.float32),
    mesh=_mesh,
    scratch_types=[
        pltpu.VMEM((KD, CD), jnp.int32),
        pltpu.VMEM((KD, CD), jnp.float32),
        pltpu.VMEM((SL,), jnp.float32),
        pltpu.VMEM_SHARED((NP,), jnp.float32),
    ],
)
def _deg_kernel(dst_hbm, ew_hbm, out_hbm, dst_v, ew_v, z_v, deg_sh):
    c = lax.axis_index("c")
    s = lax.axis_index("s")
    w = s * 2 + c
    base = s * SL

    @pl.loop(0, SL, step=16)
    def _(i):
        z_v[pl.ds(i, 16)] = jnp.zeros((16,), jnp.float32)

    pltpu.sync_copy(z_v, deg_sh.at[pl.ds(base, SL)])
    pltpu.sync_copy(dst_hbm.at[w], dst_v)
    pltpu.sync_copy(ew_hbm.at[w], ew_v)
    plsc.subcore_barrier()

    @pl.loop(0, KD)
    def _(j):
        pltpu.sync_copy(ew_v.at[j], deg_sh.at[dst_v.at[j]], add=True)

    plsc.subcore_barrier()
    pltpu.sync_copy(deg_sh.at[pl.ds(base, SL)], out_hbm.at[c, pl.ds(base, SL)])


def _hs_body(x_ref, w_ref, deg_ref, hs_ref):
    deg = deg_ref[0] + deg_ref[1] + 1.0
    dinv = lax.rsqrt(deg)
    h = jnp.dot(x_ref[...], w_ref[...], preferred_element_type=jnp.float32)
    hs_ref[...] = h * dinv[:, None]


_hs_call = pl.pallas_call(
    _hs_body,
    grid=(NB,),
    in_specs=[
        pl.BlockSpec((BLK, D), lambda i: (i, 0)),
        pl.BlockSpec((D, D), lambda i: (0, 0)),
        pl.BlockSpec((2, BLK), lambda i: (0, i)),
    ],
    out_specs=pl.BlockSpec((BLK, D), lambda i: (i, 0)),
    out_shape=jax.ShapeDtypeStruct((NP, D), jnp.float32),
)


@functools.partial(
    pl.kernel,
    out_type=jax.ShapeDtypeStruct((2, NP, D), jnp.float32),
    mesh=_mesh,
    scratch_types=[
        pltpu.VMEM((GK, C), jnp.int32),
        pltpu.VMEM((GK, C), jnp.int32),
        pltpu.VMEM((GK, C), jnp.float32),
        pltpu.VMEM((C, D), jnp.float32),
        pltpu.VMEM((C, D), jnp.float32),
        pltpu.VMEM((C, D), jnp.float32),
        pltpu.VMEM((C, D), jnp.float32),
        pltpu.VMEM_SHARED((NP, D), jnp.float32),
        pltpu.SemaphoreType.DMA,
        pltpu.SemaphoreType.DMA,
        pltpu.SemaphoreType.DMA,
        pltpu.SemaphoreType.DMA,
    ],
    compiler_params=_sc_params,
)
def _agg_kernel(hs_hbm, src_hbm, dst_hbm, ew_hbm, out_hbm,
                src_v, dst_v, ew_v, rba, rbb, sba, sbb, acc_sh,
                sga, sgb, ssa, ssb):
    c = lax.axis_index("c")
    s = lax.axis_index("s")
    w = s * 2 + c
    base = s * SL

    @pl.loop(0, C)
    def _(i):
        for k in range(D // 16):
            rba[i, pl.ds(k * 16, 16)] = jnp.zeros((16,), jnp.float32)

    @pl.when(c == 0)
    def _():
        pltpu.sync_copy(hs_hbm.at[pl.ds(base, SL)], acc_sh.at[pl.ds(base, SL)])

    @pl.when(c == 1)
    def _():
        for t in range(SL // C):
            pltpu.sync_copy(rba, acc_sh.at[pl.ds(base + t * C, C)])

    plsc.subcore_barrier()

    def _scale(rb, sb, j):
        @pl.loop(0, C)
        def _(i):
            wspl = plsc.load_gather(
                ew_v, [jnp.broadcast_to(j, (16,)), jnp.broadcast_to(i, (16,))])
            for k in range(D // 16):
                sl = pl.ds(k * 16, 16)
                sb[i, sl] = rb[i, sl] * wspl

    def _phase(g, j, rb, sb, sg, ss):
        pltpu.make_async_copy(hs_hbm.at[src_v.at[j]], rb, sg).wait()

        @pl.when(j >= 2)
        def _():
            pltpu.make_async_copy(sb, acc_sh.at[dst_v.at[0]], ss).wait()

        _scale(rb, sb, j)

        @pl.when(j + 2 < GK)
        def _():
            pltpu.async_copy(hs_hbm.at[src_v.at[j + 2]], rb, sg)

        pltpu.async_copy(sb, acc_sh.at[dst_v.at[j]], ss, add=True)

    @pl.loop(0, K // GK)
    def _(g):
        gb = g * GK
        pltpu.sync_copy(src_hbm.at[w, pl.ds(gb, GK)], src_v)
        pltpu.sync_copy(dst_hbm.at[w, pl.ds(gb, GK)], dst_v)
        pltpu.sync_copy(ew_hbm.at[w, pl.ds(gb, GK)], ew_v)

        pltpu.async_copy(hs_hbm.at[src_v.at[0]], rba, sga)
        pltpu.async_copy(hs_hbm.at[src_v.at[1]], rbb, sgb)

        @pl.loop(0, GK, step=2)
        def _(j):
            _phase(g, j, rba, sba, sga, ssa)
            _phase(g, j + 1, rbb, sbb, sgb, ssb)

        pltpu.make_async_copy(sba, acc_sh.at[dst_v.at[0]], ssa).wait()
        pltpu.make_async_copy(sbb, acc_sh.at[dst_v.at[0]], ssb).wait()

    plsc.subcore_barrier()
    pltpu.sync_copy(acc_sh.at[pl.ds(base, SL)], out_hbm.at[c, pl.ds(base, SL)])


def _pool_body(acc_ref, deg_ref, batch_ref, b1_ref, w2_ref, b2_ref, out_ref,
               sums_scr, cnt_scr):
    i = pl.program_id(0)

    @pl.when(i == 0)
    def _():
        sums_scr[...] = jnp.zeros_like(sums_scr)
        cnt_scr[...] = jnp.zeros_like(cnt_scr)

    a = acc_ref[0] + acc_ref[1]
    deg = deg_ref[0] + deg_ref[1] + 1.0
    dinv = lax.rsqrt(deg)
    h2 = jnp.maximum(a * dinv[:, None] + b1_ref[0][None, :], 0.0)
    b = batch_ref[0, 0]
    oh = (b[:, None] == lax.broadcasted_iota(jnp.int32, (BLK, G), 1))
    oh = oh.astype(jnp.float32)
    sums_scr[...] += lax.dot_general(
        oh, h2, (((0,), (0,)), ((), ())), preferred_element_type=jnp.float32)
    cnt_scr[...] += jnp.sum(oh, axis=0)[:, None]

    @pl.when(i == NB - 1)
    def _():
        pooled = sums_scr[...] / jnp.maximum(cnt_scr[...], 1.0)
        out_ref[...] = jnp.dot(
            pooled, w2_ref[...], preferred_element_type=jnp.float32
        ) + b2_ref[...]


_pool_call = pl.pallas_call(
    _pool_body,
    grid=(NB,),
    in_specs=[
        pl.BlockSpec((2, BLK, D), lambda i: (0, i, 0)),
        pl.BlockSpec((2, BLK), lambda i: (0, i)),
        pl.BlockSpec((1, 1, BLK), lambda i: (i, 0, 0)),
        pl.BlockSpec((1, D), lambda i: (0, 0)),
        pl.BlockSpec((D, G), lambda i: (0, 0)),
        pl.BlockSpec((1, G), lambda i: (0, 0)),
    ],
    out_specs=pl.BlockSpec((G, G), lambda i: (0, 0)),
    out_shape=jax.ShapeDtypeStruct((G, G), jnp.float32),
    scratch_shapes=[
        pltpu.VMEM((G, D), jnp.float32),
        pltpu.VMEM((G, 1), jnp.float32),
    ],
)


def kernel(x, edge_index, edge_attr, batch, W1, b1, W2, b2):
    src = edge_index[0]
    dst = edge_index[1]
    npad = EPAD - E
    pad_idx = jnp.arange(npad, dtype=jnp.int32) % N
    srcp = jnp.concatenate([src, pad_idx]).reshape(NW, K, C)
    dstp = jnp.concatenate([dst, pad_idx]).reshape(NW, K, C)
    ewp = jnp.concatenate(
        [edge_attr, jnp.zeros((npad,), jnp.float32)]).reshape(NW, K, C)

    xp = jnp.pad(x, ((0, NP - N), (0, 0)))
    batch_p = jnp.pad(batch, (0, NP - N), constant_values=G).reshape(NB, 1, BLK)

    deg2 = _deg_kernel(dstp.reshape(NW, KD, CD), ewp.reshape(NW, KD, CD))
    hs = _hs_call(xp, W1, deg2)
    acc = _agg_kernel(hs, srcp, dstp, ewp)
    out = _pool_call(acc, deg2, batch_p, b1.reshape(1, D), W2,
                     b2.reshape(1, G))
    return out


# speedup vs baseline: 1.8012x; 1.8012x over previous
import functools

import jax
import jax.numpy as jnp
from jax import lax
from jax.experimental import pallas as pl
from jax.experimental.pallas import tpu as pltpu
from jax.experimental.pallas import tpu_sc as plsc

N = 10000
E = 320000
D = 128
G = 64

NP = 10240
SL = NP // 16
NW = 32
C = 64
K = 160
GK = 16
CD = 128
KD = K * C // CD
EPAD = NW * K * C
BLK = 1024
NB = NP // BLK

_mesh = plsc.VectorSubcoreMesh(core_axis_name="c", subcore_axis_name="s")
_sc_params = pltpu.CompilerParams(needs_layout_passes=False)


@functools.partial(
    pl.kernel,
    out_type=jax.ShapeDtypeStruct((2, NP), jnp.float32),
    mesh=_mesh,
    scratch_types=[
        pltpu.VMEM((KD, CD), jnp.int32),
        pltpu.VMEM((KD, CD), jnp.float32),
        pltpu.VMEM((SL,), jnp.float32),
        pltpu.VMEM_SHARED((NP,), jnp.float32),
    ],
)
def _deg_kernel(dst_hbm, ew_hbm, out_hbm, dst_v, ew_v, z_v, deg_sh):
    c = lax.axis_index("c")
    s = lax.axis_index("s")
    w = s * 2 + c
    base = s * SL

    @pl.loop(0, SL, step=16)
    def _(i):
        z_v[pl.ds(i, 16)] = jnp.zeros((16,), jnp.float32)

    pltpu.sync_copy(z_v, deg_sh.at[pl.ds(base, SL)])
    pltpu.sync_copy(dst_hbm.at[w], dst_v)
    pltpu.sync_copy(ew_hbm.at[w], ew_v)
    plsc.subcore_barrier()

    @pl.loop(0, KD)
    def _(j):
        pltpu.sync_copy(ew_v.at[j], deg_sh.at[dst_v.at[j]], add=True)

    plsc.subcore_barrier()
    pltpu.sync_copy(deg_sh.at[pl.ds(base, SL)], out_hbm.at[c, pl.ds(base, SL)])


def _hs_body(x_ref, w_ref, deg_ref, hs_ref):
    deg = deg_ref[0] + deg_ref[1] + 1.0
    dinv = lax.rsqrt(deg)
    h = jnp.dot(x_ref[...], w_ref[...], preferred_element_type=jnp.float32)
    hs_ref[...] = h * dinv[:, None]


_hs_call = pl.pallas_call(
    _hs_body,
    grid=(NB,),
    in_specs=[
        pl.BlockSpec((BLK, D), lambda i: (i, 0)),
        pl.BlockSpec((D, D), lambda i: (0, 0)),
        pl.BlockSpec((2, BLK), lambda i: (0, i)),
    ],
    out_specs=pl.BlockSpec((BLK, D), lambda i: (i, 0)),
    out_shape=jax.ShapeDtypeStruct((NP, D), jnp.float32),
)


@functools.partial(
    pl.kernel,
    out_type=jax.ShapeDtypeStruct((2, NP, D), jnp.float32),
    mesh=_mesh,
    scratch_types=[
        pltpu.VMEM((GK, C), jnp.int32),
        pltpu.VMEM((GK, C), jnp.int32),
        pltpu.VMEM((GK, C), jnp.float32),
        pltpu.VMEM((C, D), jnp.float32),
        pltpu.VMEM((C, D), jnp.float32),
        pltpu.VMEM_SHARED((NP, D), jnp.float32),
        pltpu.SemaphoreType.DMA,
        pltpu.SemaphoreType.DMA,
    ],
    compiler_params=_sc_params,
)
def _agg_kernel(hs_hbm, src_hbm, dst_hbm, ew_hbm, out_hbm,
                src_v, dst_v, ew_v, rba, rbb, acc_sh, sga, sgb):
    c = lax.axis_index("c")
    s = lax.axis_index("s")
    w = s * 2 + c
    base = s * SL

    @pl.loop(0, C)
    def _(i):
        for k in range(D // 16):
            rba[i, pl.ds(k * 16, 16)] = jnp.zeros((16,), jnp.float32)

    @pl.when(c == 0)
    def _():
        pltpu.sync_copy(hs_hbm.at[pl.ds(base, SL)], acc_sh.at[pl.ds(base, SL)])

    @pl.when(c == 1)
    def _():
        for t in range(SL // C):
            pltpu.sync_copy(rba, acc_sh.at[pl.ds(base + t * C, C)])

    plsc.subcore_barrier()

    def _process(rb, j):
        @pl.loop(0, C)
        def _(i):
            wspl = plsc.load_gather(
                ew_v, [jnp.broadcast_to(j, (16,)), jnp.broadcast_to(i, (16,))])
            for k in range(D // 16):
                sl = pl.ds(k * 16, 16)
                rb[i, sl] = rb[i, sl] * wspl
        pltpu.sync_copy(rb, acc_sh.at[dst_v.at[j]], add=True)

    @pl.loop(0, K // GK)
    def _(g):
        gb = g * GK
        pltpu.sync_copy(src_hbm.at[w, pl.ds(gb, GK)], src_v)
        pltpu.sync_copy(dst_hbm.at[w, pl.ds(gb, GK)], dst_v)
        pltpu.sync_copy(ew_hbm.at[w, pl.ds(gb, GK)], ew_v)

        pltpu.async_copy(hs_hbm.at[src_v.at[0]], rba, sga)

        @pl.loop(0, GK, step=2)
        def _(j):
            pltpu.async_copy(hs_hbm.at[src_v.at[j + 1]], rbb, sgb)
            pltpu.make_async_copy(hs_hbm.at[src_v.at[j]], rba, sga).wait()
            _process(rba, j)

            @pl.when(j + 2 < GK)
            def _():
                pltpu.async_copy(hs_hbm.at[src_v.at[j + 2]], rba, sga)

            pltpu.make_async_copy(hs_hbm.at[src_v.at[j + 1]], rbb, sgb).wait()
            _process(rbb, j + 1)

    plsc.subcore_barrier()
    pltpu.sync_copy(acc_sh.at[pl.ds(base, SL)], out_hbm.at[c, pl.ds(base, SL)])


def _pool_body(acc_ref, deg_ref, batch_ref, b1_ref, w2_ref, b2_ref, out_ref,
               sums_scr, cnt_scr):
    i = pl.program_id(0)

    @pl.when(i == 0)
    def _():
        sums_scr[...] = jnp.zeros_like(sums_scr)
        cnt_scr[...] = jnp.zeros_like(cnt_scr)

    a = acc_ref[0] + acc_ref[1]
    deg = deg_ref[0] + deg_ref[1] + 1.0
    dinv = lax.rsqrt(deg)
    h2 = jnp.maximum(a * dinv[:, None] + b1_ref[0][None, :], 0.0)
    b = batch_ref[0, 0]
    oh = (b[:, None] == lax.broadcasted_iota(jnp.int32, (BLK, G), 1))
    oh = oh.astype(jnp.float32)
    sums_scr[...] += lax.dot_general(
        oh, h2, (((0,), (0,)), ((), ())), preferred_element_type=jnp.float32)
    cnt_scr[...] += jnp.sum(oh, axis=0)[:, None]

    @pl.when(i == NB - 1)
    def _():
        pooled = sums_scr[...] / jnp.maximum(cnt_scr[...], 1.0)
        out_ref[...] = jnp.dot(
            pooled, w2_ref[...], preferred_element_type=jnp.float32
        ) + b2_ref[...]


_pool_call = pl.pallas_call(
    _pool_body,
    grid=(NB,),
    in_specs=[
        pl.BlockSpec((2, BLK, D), lambda i: (0, i, 0)),
        pl.BlockSpec((2, BLK), lambda i: (0, i)),
        pl.BlockSpec((1, 1, BLK), lambda i: (i, 0, 0)),
        pl.BlockSpec((1, D), lambda i: (0, 0)),
        pl.BlockSpec((D, G), lambda i: (0, 0)),
        pl.BlockSpec((1, G), lambda i: (0, 0)),
    ],
    out_specs=pl.BlockSpec((G, G), lambda i: (0, 0)),
    out_shape=jax.ShapeDtypeStruct((G, G), jnp.float32),
    scratch_shapes=[
        pltpu.VMEM((G, D), jnp.float32),
        pltpu.VMEM((G, 1), jnp.float32),
    ],
)


def kernel(x, edge_index, edge_attr, batch, W1, b1, W2, b2):
    src = edge_index[0]
    dst = edge_index[1]
    npad = EPAD - E
    pad_idx = jnp.arange(npad, dtype=jnp.int32) % N
    srcp = jnp.concatenate([src, pad_idx]).reshape(NW, K, C)
    dstp = jnp.concatenate([dst, pad_idx]).reshape(NW, K, C)
    ewp = jnp.concatenate(
        [edge_attr, jnp.zeros((npad,), jnp.float32)]).reshape(NW, K, C)

    xp = jnp.pad(x, ((0, NP - N), (0, 0)))
    batch_p = jnp.pad(batch, (0, NP - N), constant_values=G).reshape(NB, 1, BLK)

    deg2 = _deg_kernel(dstp.reshape(NW, KD, CD), ewp.reshape(NW, KD, CD))
    hs = _hs_call(xp, W1, deg2)
    acc = _agg_kernel(hs, srcp, dstp, ewp)
    out = _pool_call(acc, deg2, batch_p, b1.reshape(1, D), W2,
                     b2.reshape(1, G))
    return out


# speedup vs baseline: 1.9300x; 1.0715x over previous
import functools

import jax
import jax.numpy as jnp
from jax import lax
from jax.experimental import pallas as pl
from jax.experimental.pallas import tpu as pltpu
from jax.experimental.pallas import tpu_sc as plsc

N = 10000
E = 320000
D = 128
G = 64

NP = 10240
SL = NP // 16
NW = 32
C = 128
K = 80
GK = 16
CD = 128
KD = K * C // CD
EPAD = NW * K * C
BLK = 1024
NB = NP // BLK

_mesh = plsc.VectorSubcoreMesh(core_axis_name="c", subcore_axis_name="s")
_sc_params = pltpu.CompilerParams(needs_layout_passes=False)


@functools.partial(
    pl.kernel,
    out_type=jax.ShapeDtypeStruct((2, NP), jnp.float32),
    mesh=_mesh,
    scratch_types=[
        pltpu.VMEM((KD, CD), jnp.int32),
        pltpu.VMEM((KD, CD), jnp.float32),
        pltpu.VMEM((SL,), jnp.float32),
        pltpu.VMEM_SHARED((NP,), jnp.float32),
    ],
)
def _deg_kernel(dst_hbm, ew_hbm, out_hbm, dst_v, ew_v, z_v, deg_sh):
    c = lax.axis_index("c")
    s = lax.axis_index("s")
    w = s * 2 + c
    base = s * SL

    @pl.loop(0, SL, step=16)
    def _(i):
        z_v[pl.ds(i, 16)] = jnp.zeros((16,), jnp.float32)

    pltpu.sync_copy(z_v, deg_sh.at[pl.ds(base, SL)])
    pltpu.sync_copy(dst_hbm.at[w], dst_v)
    pltpu.sync_copy(ew_hbm.at[w], ew_v)
    plsc.subcore_barrier()

    @pl.loop(0, KD)
    def _(j):
        pltpu.sync_copy(ew_v.at[j], deg_sh.at[dst_v.at[j]], add=True)

    plsc.subcore_barrier()
    pltpu.sync_copy(deg_sh.at[pl.ds(base, SL)], out_hbm.at[c, pl.ds(base, SL)])


def _hs_body(x_ref, w_ref, deg_ref, hs_ref):
    deg = deg_ref[0] + deg_ref[1] + 1.0
    dinv = lax.rsqrt(deg)
    h = jnp.dot(x_ref[...], w_ref[...], preferred_element_type=jnp.float32)
    hs_ref[...] = h * dinv[:, None]


_hs_call = pl.pallas_call(
    _hs_body,
    grid=(NB,),
    in_specs=[
        pl.BlockSpec((BLK, D), lambda i: (i, 0)),
        pl.BlockSpec((D, D), lambda i: (0, 0)),
        pl.BlockSpec((2, BLK), lambda i: (0, i)),
    ],
    out_specs=pl.BlockSpec((BLK, D), lambda i: (i, 0)),
    out_shape=jax.ShapeDtypeStruct((NP, D), jnp.float32),
)


@functools.partial(
    pl.kernel,
    out_type=jax.ShapeDtypeStruct((2, NP, D), jnp.float32),
    mesh=_mesh,
    scratch_types=[
        pltpu.VMEM((GK, C), jnp.int32),
        pltpu.VMEM((GK, C), jnp.int32),
        pltpu.VMEM((GK, C), jnp.float32),
        pltpu.VMEM((C, D), jnp.float32),
        pltpu.VMEM((C, D), jnp.float32),
        pltpu.VMEM_SHARED((NP, D), jnp.float32),
        pltpu.SemaphoreType.DMA,
        pltpu.SemaphoreType.DMA,
    ],
    compiler_params=_sc_params,
)
def _agg_kernel(hs_hbm, src_hbm, dst_hbm, ew_hbm, out_hbm,
                src_v, dst_v, ew_v, rba, rbb, acc_sh, sga, sgb):
    c = lax.axis_index("c")
    s = lax.axis_index("s")
    w = s * 2 + c
    base = s * SL

    @pl.loop(0, C)
    def _(i):
        for k in range(D // 16):
            rba[i, pl.ds(k * 16, 16)] = jnp.zeros((16,), jnp.float32)

    @pl.when(c == 0)
    def _():
        pltpu.sync_copy(hs_hbm.at[pl.ds(base, SL)], acc_sh.at[pl.ds(base, SL)])

    @pl.when(c == 1)
    def _():
        for t in range(SL // C):
            pltpu.sync_copy(rba, acc_sh.at[pl.ds(base + t * C, C)])

    plsc.subcore_barrier()

    def _process(rb, j):
        @pl.loop(0, C)
        def _(i):
            wspl = plsc.load_gather(
                ew_v, [jnp.broadcast_to(j, (16,)), jnp.broadcast_to(i, (16,))])
            for k in range(D // 16):
                sl = pl.ds(k * 16, 16)
                rb[i, sl] = rb[i, sl] * wspl
        pltpu.sync_copy(rb, acc_sh.at[dst_v.at[j]], add=True)

    @pl.loop(0, K // GK)
    def _(g):
        gb = g * GK
        pltpu.sync_copy(src_hbm.at[w, pl.ds(gb, GK)], src_v)
        pltpu.sync_copy(dst_hbm.at[w, pl.ds(gb, GK)], dst_v)
        pltpu.sync_copy(ew_hbm.at[w, pl.ds(gb, GK)], ew_v)

        pltpu.async_copy(hs_hbm.at[src_v.at[0]], rba, sga)

        @pl.loop(0, GK, step=2)
        def _(j):
            pltpu.async_copy(hs_hbm.at[src_v.at[j + 1]], rbb, sgb)
            pltpu.make_async_copy(hs_hbm.at[src_v.at[j]], rba, sga).wait()
            _process(rba, j)

            @pl.when(j + 2 < GK)
            def _():
                pltpu.async_copy(hs_hbm.at[src_v.at[j + 2]], rba, sga)

            pltpu.make_async_copy(hs_hbm.at[src_v.at[j + 1]], rbb, sgb).wait()
            _process(rbb, j + 1)

    plsc.subcore_barrier()
    pltpu.sync_copy(acc_sh.at[pl.ds(base, SL)], out_hbm.at[c, pl.ds(base, SL)])


def _pool_body(acc_ref, deg_ref, batch_ref, b1_ref, w2_ref, b2_ref, out_ref,
               sums_scr, cnt_scr):
    i = pl.program_id(0)

    @pl.when(i == 0)
    def _():
        sums_scr[...] = jnp.zeros_like(sums_scr)
        cnt_scr[...] = jnp.zeros_like(cnt_scr)

    a = acc_ref[0] + acc_ref[1]
    deg = deg_ref[0] + deg_ref[1] + 1.0
    dinv = lax.rsqrt(deg)
    h2 = jnp.maximum(a * dinv[:, None] + b1_ref[0][None, :], 0.0)
    b = batch_ref[0, 0]
    oh = (b[:, None] == lax.broadcasted_iota(jnp.int32, (BLK, G), 1))
    oh = oh.astype(jnp.float32)
    sums_scr[...] += lax.dot_general(
        oh, h2, (((0,), (0,)), ((), ())), preferred_element_type=jnp.float32)
    cnt_scr[...] += jnp.sum(oh, axis=0)[:, None]

    @pl.when(i == NB - 1)
    def _():
        pooled = sums_scr[...] / jnp.maximum(cnt_scr[...], 1.0)
        out_ref[...] = jnp.dot(
            pooled, w2_ref[...], preferred_element_type=jnp.float32
        ) + b2_ref[...]


_pool_call = pl.pallas_call(
    _pool_body,
    grid=(NB,),
    in_specs=[
        pl.BlockSpec((2, BLK, D), lambda i: (0, i, 0)),
        pl.BlockSpec((2, BLK), lambda i: (0, i)),
        pl.BlockSpec((1, 1, BLK), lambda i: (i, 0, 0)),
        pl.BlockSpec((1, D), lambda i: (0, 0)),
        pl.BlockSpec((D, G), lambda i: (0, 0)),
        pl.BlockSpec((1, G), lambda i: (0, 0)),
    ],
    out_specs=pl.BlockSpec((G, G), lambda i: (0, 0)),
    out_shape=jax.ShapeDtypeStruct((G, G), jnp.float32),
    scratch_shapes=[
        pltpu.VMEM((G, D), jnp.float32),
        pltpu.VMEM((G, 1), jnp.float32),
    ],
)


def kernel(x, edge_index, edge_attr, batch, W1, b1, W2, b2):
    src = edge_index[0]
    dst = edge_index[1]
    npad = EPAD - E
    pad_idx = jnp.arange(npad, dtype=jnp.int32) % N
    srcp = jnp.concatenate([src, pad_idx]).reshape(NW, K, C)
    dstp = jnp.concatenate([dst, pad_idx]).reshape(NW, K, C)
    ewp = jnp.concatenate(
        [edge_attr, jnp.zeros((npad,), jnp.float32)]).reshape(NW, K, C)

    xp = jnp.pad(x, ((0, NP - N), (0, 0)))
    batch_p = jnp.pad(batch, (0, NP - N), constant_values=G).reshape(NB, 1, BLK)

    deg2 = _deg_kernel(dstp.reshape(NW, KD, CD), ewp.reshape(NW, KD, CD))
    hs = _hs_call(xp, W1, deg2)
    acc = _agg_kernel(hs, srcp, dstp, ewp)
    out = _pool_call(acc, deg2, batch_p, b1.reshape(1, D), W2,
                     b2.reshape(1, G))
    return out


# speedup vs baseline: 2.1896x; 1.1345x over previous
import functools

import jax
import jax.numpy as jnp
from jax import lax
from jax.experimental import pallas as pl
from jax.experimental.pallas import tpu as pltpu
from jax.experimental.pallas import tpu_sc as plsc

N = 10000
E = 320000
D = 128
G = 64

NP = 10240
SL = NP // 16
NW = 32
C = 128
K = 80
GK = 16
CD = 128
KD = K * C // CD
EPAD = NW * K * C
BLK = 1024
NB = NP // BLK

_mesh = plsc.VectorSubcoreMesh(core_axis_name="c", subcore_axis_name="s")
_sc_params = pltpu.CompilerParams(needs_layout_passes=False)


@functools.partial(
    pl.kernel,
    out_type=jax.ShapeDtypeStruct((2, NP), jnp.float32),
    mesh=_mesh,
    scratch_types=[
        pltpu.VMEM((KD, CD), jnp.int32),
        pltpu.VMEM((KD, CD), jnp.float32),
        pltpu.VMEM((SL,), jnp.float32),
        pltpu.VMEM_SHARED((NP,), jnp.float32),
    ],
)
def _deg_kernel(dst_hbm, ew_hbm, out_hbm, dst_v, ew_v, z_v, deg_sh):
    c = lax.axis_index("c")
    s = lax.axis_index("s")
    w = s * 2 + c
    base = s * SL

    @pl.loop(0, SL, step=16)
    def _(i):
        z_v[pl.ds(i, 16)] = jnp.zeros((16,), jnp.float32)

    pltpu.sync_copy(z_v, deg_sh.at[pl.ds(base, SL)])
    pltpu.sync_copy(dst_hbm.at[w], dst_v)
    pltpu.sync_copy(ew_hbm.at[w], ew_v)
    plsc.subcore_barrier()

    @pl.loop(0, KD)
    def _(j):
        pltpu.sync_copy(ew_v.at[j], deg_sh.at[dst_v.at[j]], add=True)

    plsc.subcore_barrier()
    pltpu.sync_copy(deg_sh.at[pl.ds(base, SL)], out_hbm.at[c, pl.ds(base, SL)])


def _hs_body(x_ref, w_ref, deg_ref, hs_ref):
    deg = deg_ref[0] + deg_ref[1] + 1.0
    dinv = lax.rsqrt(deg)
    h = jnp.dot(x_ref[...], w_ref[...], preferred_element_type=jnp.float32)
    hs_ref[...] = h * dinv[:, None]


_hs_call = pl.pallas_call(
    _hs_body,
    grid=(NB,),
    in_specs=[
        pl.BlockSpec((BLK, D), lambda i: (i, 0)),
        pl.BlockSpec((D, D), lambda i: (0, 0)),
        pl.BlockSpec((2, BLK), lambda i: (0, i)),
    ],
    out_specs=pl.BlockSpec((BLK, D), lambda i: (i, 0)),
    out_shape=jax.ShapeDtypeStruct((NP, D), jnp.float32),
)


@functools.partial(
    pl.kernel,
    out_type=jax.ShapeDtypeStruct((2, NP, D), jnp.float32),
    mesh=_mesh,
    scratch_types=[
        pltpu.VMEM((GK, C), jnp.int32),
        pltpu.VMEM((GK, C), jnp.int32),
        pltpu.VMEM((GK, C), jnp.float32),
        pltpu.VMEM((C, D), jnp.float32),
        pltpu.VMEM((C, D), jnp.float32),
        pltpu.VMEM_SHARED((NP, D), jnp.float32),
        pltpu.SemaphoreType.DMA,
        pltpu.SemaphoreType.DMA,
    ],
    compiler_params=_sc_params,
)
def _agg_kernel(hs_hbm, src_hbm, dst_hbm, ew_hbm, out_hbm,
                src_v, dst_v, ew_v, rba, rbb, acc_sh, sga, sgb):
    c = lax.axis_index("c")
    s = lax.axis_index("s")
    w = s * 2 + c
    base = s * SL

    @pl.loop(0, C)
    def _(i):
        for k in range(D // 16):
            rba[i, pl.ds(k * 16, 16)] = jnp.zeros((16,), jnp.float32)

    @pl.when(c == 0)
    def _():
        pltpu.sync_copy(hs_hbm.at[pl.ds(base, SL)], acc_sh.at[pl.ds(base, SL)])

    @pl.when(c == 1)
    def _():
        for t in range(SL // C):
            pltpu.sync_copy(rba, acc_sh.at[pl.ds(base + t * C, C)])

    plsc.subcore_barrier()

    def _process(rb, j):
        @pl.loop(0, C, step=2)
        def _(i):
            jb = jnp.broadcast_to(j, (16,))
            w0 = plsc.load_gather(ew_v, [jb, jnp.broadcast_to(i, (16,))])
            w1 = plsc.load_gather(ew_v, [jb, jnp.broadcast_to(i + 1, (16,))])
            for k in range(D // 16):
                sl = pl.ds(k * 16, 16)
                rb[i, sl] = rb[i, sl] * w0
                rb[i + 1, sl] = rb[i + 1, sl] * w1
        pltpu.sync_copy(rb, acc_sh.at[dst_v.at[j]], add=True)

    @pl.loop(0, K // GK)
    def _(g):
        gb = g * GK
        pltpu.sync_copy(src_hbm.at[w, pl.ds(gb, GK)], src_v)
        pltpu.sync_copy(dst_hbm.at[w, pl.ds(gb, GK)], dst_v)
        pltpu.sync_copy(ew_hbm.at[w, pl.ds(gb, GK)], ew_v)

        pltpu.async_copy(hs_hbm.at[src_v.at[0]], rba, sga)

        @pl.loop(0, GK, step=2)
        def _(j):
            pltpu.async_copy(hs_hbm.at[src_v.at[j + 1]], rbb, sgb)
            pltpu.make_async_copy(hs_hbm.at[src_v.at[j]], rba, sga).wait()
            _process(rba, j)

            @pl.when(j + 2 < GK)
            def _():
                pltpu.async_copy(hs_hbm.at[src_v.at[j + 2]], rba, sga)

            pltpu.make_async_copy(hs_hbm.at[src_v.at[j + 1]], rbb, sgb).wait()
            _process(rbb, j + 1)

    plsc.subcore_barrier()
    pltpu.sync_copy(acc_sh.at[pl.ds(base, SL)], out_hbm.at[c, pl.ds(base, SL)])


def _pool_body(acc_ref, deg_ref, batch_ref, b1_ref, w2_ref, b2_ref, out_ref,
               sums_scr, cnt_scr):
    i = pl.program_id(0)

    @pl.when(i == 0)
    def _():
        sums_scr[...] = jnp.zeros_like(sums_scr)
        cnt_scr[...] = jnp.zeros_like(cnt_scr)

    a = acc_ref[0] + acc_ref[1]
    deg = deg_ref[0] + deg_ref[1] + 1.0
    dinv = lax.rsqrt(deg)
    h2 = jnp.maximum(a * dinv[:, None] + b1_ref[0][None, :], 0.0)
    b = batch_ref[0, 0]
    oh = (b[:, None] == lax.broadcasted_iota(jnp.int32, (BLK, G), 1))
    oh = oh.astype(jnp.float32)
    sums_scr[...] += lax.dot_general(
        oh, h2, (((0,), (0,)), ((), ())), preferred_element_type=jnp.float32)
    cnt_scr[...] += jnp.sum(oh, axis=0)[:, None]

    @pl.when(i == NB - 1)
    def _():
        pooled = sums_scr[...] / jnp.maximum(cnt_scr[...], 1.0)
        out_ref[...] = jnp.dot(
            pooled, w2_ref[...], preferred_element_type=jnp.float32
        ) + b2_ref[...]


_pool_call = pl.pallas_call(
    _pool_body,
    grid=(NB,),
    in_specs=[
        pl.BlockSpec((2, BLK, D), lambda i: (0, i, 0)),
        pl.BlockSpec((2, BLK), lambda i: (0, i)),
        pl.BlockSpec((1, 1, BLK), lambda i: (i, 0, 0)),
        pl.BlockSpec((1, D), lambda i: (0, 0)),
        pl.BlockSpec((D, G), lambda i: (0, 0)),
        pl.BlockSpec((1, G), lambda i: (0, 0)),
    ],
    out_specs=pl.BlockSpec((G, G), lambda i: (0, 0)),
    out_shape=jax.ShapeDtypeStruct((G, G), jnp.float32),
    scratch_shapes=[
        pltpu.VMEM((G, D), jnp.float32),
        pltpu.VMEM((G, 1), jnp.float32),
    ],
)


def kernel(x, edge_index, edge_attr, batch, W1, b1, W2, b2):
    src = edge_index[0]
    dst = edge_index[1]
    npad = EPAD - E
    pad_idx = jnp.arange(npad, dtype=jnp.int32) % N
    srcp = jnp.concatenate([src, pad_idx]).reshape(NW, K, C)
    dstp = jnp.concatenate([dst, pad_idx]).reshape(NW, K, C)
    ewp = jnp.concatenate(
        [edge_attr, jnp.zeros((npad,), jnp.float32)]).reshape(NW, K, C)

    xp = jnp.pad(x, ((0, NP - N), (0, 0)))
    batch_p = jnp.pad(batch, (0, NP - N), constant_values=G).reshape(NB, 1, BLK)

    deg2 = _deg_kernel(dstp.reshape(NW, KD, CD), ewp.reshape(NW, KD, CD))
    hs = _hs_call(xp, W1, deg2)
    acc = _agg_kernel(hs, srcp, dstp, ewp)
    out = _pool_call(acc, deg2, batch_p, b1.reshape(1, D), W2,
                     b2.reshape(1, G))
    return out


# speedup vs baseline: 2.2479x; 1.0266x over previous
import functools

import jax
import jax.numpy as jnp
from jax import lax
from jax.experimental import pallas as pl
from jax.experimental.pallas import tpu as pltpu
from jax.experimental.pallas import tpu_sc as plsc

N = 10000
E = 320000
D = 128
G = 64

NP = 10240
SL = NP // 16
NW = 32
C = 128
K = 80
GK = 16
CD = 128
KD = K * C // CD
EPAD = NW * K * C
BLK = 1024
NB = NP // BLK

_mesh = plsc.VectorSubcoreMesh(core_axis_name="c", subcore_axis_name="s")
_sc_params = pltpu.CompilerParams(needs_layout_passes=False)


@functools.partial(
    pl.kernel,
    out_type=jax.ShapeDtypeStruct((2, NP), jnp.float32),
    mesh=_mesh,
    scratch_types=[
        pltpu.VMEM((KD, CD), jnp.int32),
        pltpu.VMEM((KD, CD), jnp.float32),
        pltpu.VMEM((SL,), jnp.float32),
        pltpu.VMEM_SHARED((NP,), jnp.float32),
    ],
)
def _deg_kernel(dst_hbm, ew_hbm, out_hbm, dst_v, ew_v, z_v, deg_sh):
    c = lax.axis_index("c")
    s = lax.axis_index("s")
    w = s * 2 + c
    base = s * SL

    @pl.loop(0, SL, step=16)
    def _(i):
        z_v[pl.ds(i, 16)] = jnp.zeros((16,), jnp.float32)

    pltpu.sync_copy(z_v, deg_sh.at[pl.ds(base, SL)])
    pltpu.sync_copy(dst_hbm.at[w], dst_v)
    pltpu.sync_copy(ew_hbm.at[w], ew_v)
    plsc.subcore_barrier()

    @pl.loop(0, KD)
    def _(j):
        pltpu.sync_copy(ew_v.at[j], deg_sh.at[dst_v.at[j]], add=True)

    plsc.subcore_barrier()
    pltpu.sync_copy(deg_sh.at[pl.ds(base, SL)], out_hbm.at[c, pl.ds(base, SL)])


def _hs_body(x_ref, w_ref, deg_ref, hs_ref):
    deg = deg_ref[0] + deg_ref[1] + 1.0
    dinv = lax.rsqrt(deg)
    h = jnp.dot(x_ref[...], w_ref[...], preferred_element_type=jnp.float32)
    hs_ref[...] = h * dinv[:, None]


_hs_call = pl.pallas_call(
    _hs_body,
    grid=(NB,),
    in_specs=[
        pl.BlockSpec((BLK, D), lambda i: (i, 0)),
        pl.BlockSpec((D, D), lambda i: (0, 0)),
        pl.BlockSpec((2, BLK), lambda i: (0, i)),
    ],
    out_specs=pl.BlockSpec((BLK, D), lambda i: (i, 0)),
    out_shape=jax.ShapeDtypeStruct((NP, D), jnp.float32),
)


@functools.partial(
    pl.kernel,
    out_type=jax.ShapeDtypeStruct((2, NP, D), jnp.float32),
    mesh=_mesh,
    scratch_types=[
        pltpu.VMEM((GK, C), jnp.int32),
        pltpu.VMEM((GK, C), jnp.int32),
        pltpu.VMEM((GK, C), jnp.float32),
        pltpu.VMEM((C, D), jnp.float32),
        pltpu.VMEM((C, D), jnp.float32),
        pltpu.VMEM_SHARED((NP, D), jnp.float32),
        pltpu.SemaphoreType.DMA,
        pltpu.SemaphoreType.DMA,
    ],
    compiler_params=_sc_params,
)
def _agg_kernel(hs_hbm, src_hbm, dst_hbm, ew_hbm, out_hbm,
                src_v, dst_v, ew_v, rba, rbb, acc_sh, sga, sgb):
    c = lax.axis_index("c")
    s = lax.axis_index("s")
    w = s * 2 + c
    base = s * SL

    @pl.loop(0, C)
    def _(i):
        for k in range(D // 16):
            rba[i, pl.ds(k * 16, 16)] = jnp.zeros((16,), jnp.float32)

    @pl.when(c == 0)
    def _():
        pltpu.sync_copy(hs_hbm.at[pl.ds(base, SL)], acc_sh.at[pl.ds(base, SL)])

    @pl.when(c == 1)
    def _():
        for t in range(SL // C):
            pltpu.sync_copy(rba, acc_sh.at[pl.ds(base + t * C, C)])

    plsc.subcore_barrier()

    def _process(rb, j):
        @pl.loop(0, C, step=4)
        def _(i):
            jb = jnp.broadcast_to(j, (16,))
            ws = [plsc.load_gather(ew_v, [jb, jnp.broadcast_to(i + u, (16,))])
                  for u in range(4)]
            for k in range(D // 16):
                sl = pl.ds(k * 16, 16)
                for u in range(4):
                    rb[i + u, sl] = rb[i + u, sl] * ws[u]
        pltpu.sync_copy(rb, acc_sh.at[dst_v.at[j]], add=True)

    @pl.loop(0, K // GK)
    def _(g):
        gb = g * GK
        pltpu.sync_copy(src_hbm.at[w, pl.ds(gb, GK)], src_v)
        pltpu.sync_copy(dst_hbm.at[w, pl.ds(gb, GK)], dst_v)
        pltpu.sync_copy(ew_hbm.at[w, pl.ds(gb, GK)], ew_v)

        pltpu.async_copy(hs_hbm.at[src_v.at[0]], rba, sga)

        @pl.loop(0, GK, step=2)
        def _(j):
            pltpu.async_copy(hs_hbm.at[src_v.at[j + 1]], rbb, sgb)
            pltpu.make_async_copy(hs_hbm.at[src_v.at[j]], rba, sga).wait()
            _process(rba, j)

            @pl.when(j + 2 < GK)
            def _():
                pltpu.async_copy(hs_hbm.at[src_v.at[j + 2]], rba, sga)

            pltpu.make_async_copy(hs_hbm.at[src_v.at[j + 1]], rbb, sgb).wait()
            _process(rbb, j + 1)

    plsc.subcore_barrier()
    pltpu.sync_copy(acc_sh.at[pl.ds(base, SL)], out_hbm.at[c, pl.ds(base, SL)])


def _pool_body(acc_ref, deg_ref, batch_ref, b1_ref, w2_ref, b2_ref, out_ref,
               sums_scr, cnt_scr):
    i = pl.program_id(0)

    @pl.when(i == 0)
    def _():
        sums_scr[...] = jnp.zeros_like(sums_scr)
        cnt_scr[...] = jnp.zeros_like(cnt_scr)

    a = acc_ref[0] + acc_ref[1]
    deg = deg_ref[0] + deg_ref[1] + 1.0
    dinv = lax.rsqrt(deg)
    h2 = jnp.maximum(a * dinv[:, None] + b1_ref[0][None, :], 0.0)
    b = batch_ref[0, 0]
    oh = (b[:, None] == lax.broadcasted_iota(jnp.int32, (BLK, G), 1))
    oh = oh.astype(jnp.float32)
    sums_scr[...] += lax.dot_general(
        oh, h2, (((0,), (0,)), ((), ())), preferred_element_type=jnp.float32)
    cnt_scr[...] += jnp.sum(oh, axis=0)[:, None]

    @pl.when(i == NB - 1)
    def _():
        pooled = sums_scr[...] / jnp.maximum(cnt_scr[...], 1.0)
        out_ref[...] = jnp.dot(
            pooled, w2_ref[...], preferred_element_type=jnp.float32
        ) + b2_ref[...]


_pool_call = pl.pallas_call(
    _pool_body,
    grid=(NB,),
    in_specs=[
        pl.BlockSpec((2, BLK, D), lambda i: (0, i, 0)),
        pl.BlockSpec((2, BLK), lambda i: (0, i)),
        pl.BlockSpec((1, 1, BLK), lambda i: (i, 0, 0)),
        pl.BlockSpec((1, D), lambda i: (0, 0)),
        pl.BlockSpec((D, G), lambda i: (0, 0)),
        pl.BlockSpec((1, G), lambda i: (0, 0)),
    ],
    out_specs=pl.BlockSpec((G, G), lambda i: (0, 0)),
    out_shape=jax.ShapeDtypeStruct((G, G), jnp.float32),
    scratch_shapes=[
        pltpu.VMEM((G, D), jnp.float32),
        pltpu.VMEM((G, 1), jnp.float32),
    ],
)


def kernel(x, edge_index, edge_attr, batch, W1, b1, W2, b2):
    src = edge_index[0]
    dst = edge_index[1]
    npad = EPAD - E
    pad_idx = jnp.arange(npad, dtype=jnp.int32) % N
    srcp = jnp.concatenate([src, pad_idx]).reshape(NW, K, C)
    dstp = jnp.concatenate([dst, pad_idx]).reshape(NW, K, C)
    ewp = jnp.concatenate(
        [edge_attr, jnp.zeros((npad,), jnp.float32)]).reshape(NW, K, C)

    xp = jnp.pad(x, ((0, NP - N), (0, 0)))
    batch_p = jnp.pad(batch, (0, NP - N), constant_values=G).reshape(NB, 1, BLK)

    deg2 = _deg_kernel(dstp.reshape(NW, KD, CD), ewp.reshape(NW, KD, CD))
    hs = _hs_call(xp, W1, deg2)
    acc = _agg_kernel(hs, srcp, dstp, ewp)
    out = _pool_call(acc, deg2, batch_p, b1.reshape(1, D), W2,
                     b2.reshape(1, G))
    return out


# speedup vs baseline: 2.3020x; 1.0240x over previous
import functools

import jax
import jax.numpy as jnp
from jax import lax
from jax.experimental import pallas as pl
from jax.experimental.pallas import tpu as pltpu
from jax.experimental.pallas import tpu_sc as plsc

N = 10000
E = 320000
D = 128
G = 64

NP = 10240
SL = NP // 16
NW = 32
C = 128
K = 80
GK = 16
CD = 128
KD = K * C // CD
EPAD = NW * K * C
BLK = 1024
NB = NP // BLK

_mesh = plsc.VectorSubcoreMesh(core_axis_name="c", subcore_axis_name="s")
_sc_params = pltpu.CompilerParams(needs_layout_passes=False)


@functools.partial(
    pl.kernel,
    out_type=jax.ShapeDtypeStruct((2, NP), jnp.float32),
    mesh=_mesh,
    scratch_types=[
        pltpu.VMEM((KD, CD), jnp.int32),
        pltpu.VMEM((KD, CD), jnp.float32),
        pltpu.VMEM((SL,), jnp.float32),
        pltpu.VMEM_SHARED((NP,), jnp.float32),
        pltpu.SemaphoreType.DMA,
    ],
)
def _deg_kernel(dst_hbm, ew_hbm, out_hbm, dst_v, ew_v, z_v, deg_sh, sd):
    c = lax.axis_index("c")
    s = lax.axis_index("s")
    w = s * 2 + c
    base = s * SL

    @pl.loop(0, SL, step=16)
    def _(i):
        z_v[pl.ds(i, 16)] = jnp.zeros((16,), jnp.float32)

    pltpu.sync_copy(z_v, deg_sh.at[pl.ds(base, SL)])
    pltpu.sync_copy(dst_hbm.at[w], dst_v)
    pltpu.sync_copy(ew_hbm.at[w], ew_v)
    plsc.subcore_barrier()

    @pl.loop(0, KD, step=8)
    def _(j):
        for u in range(8):
            pltpu.async_copy(ew_v.at[j + u], deg_sh.at[dst_v.at[j + u]], sd,
                             add=True)
        for u in range(8):
            pltpu.make_async_copy(ew_v.at[0], deg_sh.at[dst_v.at[0]],
                                  sd).wait()

    plsc.subcore_barrier()
    pltpu.sync_copy(deg_sh.at[pl.ds(base, SL)], out_hbm.at[c, pl.ds(base, SL)])


def _hs_body(x_ref, w_ref, deg_ref, hs_ref):
    deg = deg_ref[0] + deg_ref[1] + 1.0
    dinv = lax.rsqrt(deg)
    h = jnp.dot(x_ref[...], w_ref[...], preferred_element_type=jnp.float32)
    hs_ref[...] = h * dinv[:, None]


_hs_call = pl.pallas_call(
    _hs_body,
    grid=(NB,),
    in_specs=[
        pl.BlockSpec((BLK, D), lambda i: (i, 0)),
        pl.BlockSpec((D, D), lambda i: (0, 0)),
        pl.BlockSpec((2, BLK), lambda i: (0, i)),
    ],
    out_specs=pl.BlockSpec((BLK, D), lambda i: (i, 0)),
    out_shape=jax.ShapeDtypeStruct((NP, D), jnp.float32),
)


@functools.partial(
    pl.kernel,
    out_type=jax.ShapeDtypeStruct((2, NP, D), jnp.float32),
    mesh=_mesh,
    scratch_types=[
        pltpu.VMEM((GK, C), jnp.int32),
        pltpu.VMEM((GK, C), jnp.int32),
        pltpu.VMEM((GK, C), jnp.float32),
        pltpu.VMEM((C, D), jnp.float32),
        pltpu.VMEM((C, D), jnp.float32),
        pltpu.VMEM_SHARED((NP, D), jnp.float32),
        pltpu.SemaphoreType.DMA,
        pltpu.SemaphoreType.DMA,
    ],
    compiler_params=_sc_params,
)
def _agg_kernel(hs_hbm, src_hbm, dst_hbm, ew_hbm, out_hbm,
                src_v, dst_v, ew_v, rba, rbb, acc_sh, sga, sgb):
    c = lax.axis_index("c")
    s = lax.axis_index("s")
    w = s * 2 + c
    base = s * SL

    @pl.loop(0, C)
    def _(i):
        for k in range(D // 16):
            rba[i, pl.ds(k * 16, 16)] = jnp.zeros((16,), jnp.float32)

    @pl.when(c == 0)
    def _():
        pltpu.sync_copy(hs_hbm.at[pl.ds(base, SL)], acc_sh.at[pl.ds(base, SL)])

    @pl.when(c == 1)
    def _():
        for t in range(SL // C):
            pltpu.sync_copy(rba, acc_sh.at[pl.ds(base + t * C, C)])

    plsc.subcore_barrier()

    def _process(rb, j):
        @pl.loop(0, C, step=8)
        def _(i):
            jb = jnp.broadcast_to(j, (16,))
            ws = [plsc.load_gather(ew_v, [jb, jnp.broadcast_to(i + u, (16,))])
                  for u in range(8)]
            for k in range(D // 16):
                sl = pl.ds(k * 16, 16)
                for u in range(8):
                    rb[i + u, sl] = rb[i + u, sl] * ws[u]
        pltpu.sync_copy(rb, acc_sh.at[dst_v.at[j]], add=True)

    @pl.loop(0, K // GK)
    def _(g):
        gb = g * GK
        pltpu.sync_copy(src_hbm.at[w, pl.ds(gb, GK)], src_v)
        pltpu.sync_copy(dst_hbm.at[w, pl.ds(gb, GK)], dst_v)
        pltpu.sync_copy(ew_hbm.at[w, pl.ds(gb, GK)], ew_v)

        pltpu.async_copy(hs_hbm.at[src_v.at[0]], rba, sga)

        @pl.loop(0, GK, step=2)
        def _(j):
            pltpu.async_copy(hs_hbm.at[src_v.at[j + 1]], rbb, sgb)
            pltpu.make_async_copy(hs_hbm.at[src_v.at[j]], rba, sga).wait()
            _process(rba, j)

            @pl.when(j + 2 < GK)
            def _():
                pltpu.async_copy(hs_hbm.at[src_v.at[j + 2]], rba, sga)

            pltpu.make_async_copy(hs_hbm.at[src_v.at[j + 1]], rbb, sgb).wait()
            _process(rbb, j + 1)

    plsc.subcore_barrier()
    pltpu.sync_copy(acc_sh.at[pl.ds(base, SL)], out_hbm.at[c, pl.ds(base, SL)])


def _pool_body(acc_ref, deg_ref, batch_ref, b1_ref, w2_ref, b2_ref, out_ref,
               sums_scr, cnt_scr):
    i = pl.program_id(0)

    @pl.when(i == 0)
    def _():
        sums_scr[...] = jnp.zeros_like(sums_scr)
        cnt_scr[...] = jnp.zeros_like(cnt_scr)

    a = acc_ref[0] + acc_ref[1]
    deg = deg_ref[0] + deg_ref[1] + 1.0
    dinv = lax.rsqrt(deg)
    h2 = jnp.maximum(a * dinv[:, None] + b1_ref[0][None, :], 0.0)
    b = batch_ref[0, 0]
    oh = (b[:, None] == lax.broadcasted_iota(jnp.int32, (BLK, G), 1))
    oh = oh.astype(jnp.float32)
    sums_scr[...] += lax.dot_general(
        oh, h2, (((0,), (0,)), ((), ())), preferred_element_type=jnp.float32)
    cnt_scr[...] += jnp.sum(oh, axis=0)[:, None]

    @pl.when(i == NB - 1)
    def _():
        pooled = sums_scr[...] / jnp.maximum(cnt_scr[...], 1.0)
        out_ref[...] = jnp.dot(
            pooled, w2_ref[...], preferred_element_type=jnp.float32
        ) + b2_ref[...]


_pool_call = pl.pallas_call(
    _pool_body,
    grid=(NB,),
    in_specs=[
        pl.BlockSpec((2, BLK, D), lambda i: (0, i, 0)),
        pl.BlockSpec((2, BLK), lambda i: (0, i)),
        pl.BlockSpec((1, 1, BLK), lambda i: (i, 0, 0)),
        pl.BlockSpec((1, D), lambda i: (0, 0)),
        pl.BlockSpec((D, G), lambda i: (0, 0)),
        pl.BlockSpec((1, G), lambda i: (0, 0)),
    ],
    out_specs=pl.BlockSpec((G, G), lambda i: (0, 0)),
    out_shape=jax.ShapeDtypeStruct((G, G), jnp.float32),
    scratch_shapes=[
        pltpu.VMEM((G, D), jnp.float32),
        pltpu.VMEM((G, 1), jnp.float32),
    ],
)


def kernel(x, edge_index, edge_attr, batch, W1, b1, W2, b2):
    src = edge_index[0]
    dst = edge_index[1]
    npad = EPAD - E
    pad_idx = jnp.arange(npad, dtype=jnp.int32) % N
    srcp = jnp.concatenate([src, pad_idx]).reshape(NW, K, C)
    dstp = jnp.concatenate([dst, pad_idx]).reshape(NW, K, C)
    ewp = jnp.concatenate(
        [edge_attr, jnp.zeros((npad,), jnp.float32)]).reshape(NW, K, C)

    xp = jnp.pad(x, ((0, NP - N), (0, 0)))
    batch_p = jnp.pad(batch, (0, NP - N), constant_values=G).reshape(NB, 1, BLK)

    deg2 = _deg_kernel(dstp.reshape(NW, KD, CD), ewp.reshape(NW, KD, CD))
    hs = _hs_call(xp, W1, deg2)
    acc = _agg_kernel(hs, srcp, dstp, ewp)
    out = _pool_call(acc, deg2, batch_p, b1.reshape(1, D), W2,
                     b2.reshape(1, G))
    return out
